# baseline (speedup 1.0000x reference)
import functools

import jax
import jax.numpy as jnp
from jax import lax
from jax.experimental import pallas as pl
from jax.experimental.pallas import tpu as pltpu
from jax.experimental.pallas import tpu_sc as plsc

_VOCAB = 100000
_DIM = 16
_BATCH = 1024
_CTX = 50

_NC, _NS = 2, 16
_NW = _NC * _NS
_CHUNK = 128


def _make_emb_body(bpw, nchunk):
    gpw = bpw * _CTX

    def _emb_body(idx_hbm, table_hbm, out_hbm, idx_v, rows_v, acc_v, sem):
        wid = lax.axis_index("s") * _NC + lax.axis_index("c")
        pltpu.sync_copy(idx_hbm.at[wid], idx_v)
        copies = []
        for c in range(nchunk):
            copies.append(
                pltpu.make_async_copy(
                    table_hbm.at[idx_v.at[c]],
                    rows_v.at[pl.ds(c * _CHUNK, _CHUNK)],
                    sem,
                )
            )
        for cp in copies:
            cp.start()
        for cp in copies:
            cp.wait()

        def body(r, carry):
            base = r * _CTX
            acc = rows_v[base]
            for j in range(1, _CTX):
                acc = acc + rows_v[base + j]
            acc_v[r] = acc
            return carry

        lax.fori_loop(0, bpw, body, 0)
        pltpu.sync_copy(acc_v, out_hbm.at[pl.ds(wid * bpw, bpw)])

    return _emb_body


def _embed(words, table):
    nrows = words.shape[0]
    bpw = nrows // _NW
    gpw = bpw * _CTX
    nchunk = (gpw + _CHUNK - 1) // _CHUNK
    gpad = nchunk * _CHUNK
    idx = words.reshape(_NW, gpw).astype(jnp.int32)
    idx = jnp.pad(idx, ((0, 0), (0, gpad - gpw))).reshape(_NW, nchunk, _CHUNK)
    mesh = plsc.VectorSubcoreMesh(core_axis_name="c", subcore_axis_name="s")
    f = functools.partial(
        pl.kernel,
        mesh=mesh,
        out_type=jax.ShapeDtypeStruct((nrows, _DIM), jnp.float32),
        scratch_types=[
            pltpu.VMEM((nchunk, _CHUNK), jnp.int32),
            pltpu.VMEM((gpad, _DIM), jnp.float32),
            pltpu.VMEM((bpw, _DIM), jnp.float32),
            pltpu.SemaphoreType.DMA,
        ],
        compiler_params=pltpu.CompilerParams(use_tc_tiling_on_sc=False),
    )(_make_emb_body(bpw, nchunk))
    return f(idx, table)


_RB = 64
_NRSTEP = _BATCH // _RB
_NBUF = 2


def _proj_body(emb_ref, w_ref, b_ref, out_ref):
    out_ref[...] = (
        lax.dot_general(
            emb_ref[...],
            w_ref[...],
            dimension_numbers=(((1,), (0,)), ((), ())),
            preferred_element_type=jnp.float32,
        )
        + b_ref[...]
    )


def _proj_body_alias(prev_ref, emb_ref, w_ref, b_ref, out_ref):
    _proj_body(emb_ref, w_ref, b_ref, out_ref)


def _project_rows(emb, wt, b2, prev, row0):
    nrows = emb.shape[0]
    nstep = nrows // _RB
    r0 = row0 // _RB
    cp = pltpu.CompilerParams(
        dimension_semantics=("arbitrary",),
        disable_bounds_checks=True,
    )
    common = dict(
        grid=(nstep,),
        out_shape=jax.ShapeDtypeStruct((_BATCH, _VOCAB), jnp.float32),
        out_specs=pl.BlockSpec((_RB, _VOCAB), lambda i: (r0 + i, 0)),
        compiler_params=cp,
    )
    emb_spec = pl.BlockSpec((_RB, _DIM), lambda i: (i, 0))
    w_spec = pl.BlockSpec((_DIM, _VOCAB), lambda i: (0, 0))
    b_spec = pl.BlockSpec((1, _VOCAB), lambda i: (0, 0))
    if prev is None:
        return pl.pallas_call(
            _proj_body,
            in_specs=[emb_spec, w_spec, b_spec],
            **common,
        )(emb, wt, b2)
    prev_spec = pl.BlockSpec(memory_space=pltpu.MemorySpace.HBM)
    return pl.pallas_call(
        _proj_body_alias,
        in_specs=[prev_spec, emb_spec, w_spec, b_spec],
        input_output_aliases={0: 0},
        **common,
    )(prev, emb, wt, b2)


_SPLIT = 128


def kernel(words, table, W, b):
    wt = W.T
    b2 = b.reshape(1, _VOCAB)
    emb1 = _embed(words[:_SPLIT], table)
    emb2 = _embed(words[_SPLIT:], table)
    out1 = _project_rows(emb1, wt, b2, None, 0)
    out2 = _project_rows(emb2, wt, b2, out1, _SPLIT)
    return out2


# speedup vs baseline: 1.0233x; 1.0233x over previous
import functools

import jax
import jax.numpy as jnp
from jax import lax
from jax.experimental import pallas as pl
from jax.experimental.pallas import tpu as pltpu
from jax.experimental.pallas import tpu_sc as plsc

_VOCAB = 100000
_DIM = 16
_BATCH = 1024
_CTX = 50

_NC, _NS = 2, 16
_NW = _NC * _NS
_CHUNK = 128


def _make_emb_body(bpw, nchunk):
    gpw = bpw * _CTX

    def _emb_body(idx_hbm, table_hbm, out_hbm, idx_v, rows_v, acc_v, sem):
        wid = lax.axis_index("s") * _NC + lax.axis_index("c")
        pltpu.sync_copy(idx_hbm.at[wid], idx_v)
        copies = []
        for c in range(nchunk):
            copies.append(
                pltpu.make_async_copy(
                    table_hbm.at[idx_v.at[c]],
                    rows_v.at[pl.ds(c * _CHUNK, _CHUNK)],
                    sem,
                )
            )
        for cp in copies:
            cp.start()
        for cp in copies:
            cp.wait()

        def body(r, carry):
            base = r * _CTX
            acc = rows_v[base]
            for j in range(1, _CTX):
                acc = acc + rows_v[base + j]
            acc_v[r] = acc
            return carry

        lax.fori_loop(0, bpw, body, 0)
        pltpu.sync_copy(acc_v, out_hbm.at[pl.ds(wid * bpw, bpw)])

    return _emb_body


def _embed(words, table):
    nrows = words.shape[0]
    bpw = nrows // _NW
    gpw = bpw * _CTX
    nchunk = (gpw + _CHUNK - 1) // _CHUNK
    gpad = nchunk * _CHUNK
    idx = words.reshape(_NW, gpw).astype(jnp.int32)
    idx = jnp.pad(idx, ((0, 0), (0, gpad - gpw))).reshape(_NW, nchunk, _CHUNK)
    mesh = plsc.VectorSubcoreMesh(core_axis_name="c", subcore_axis_name="s")
    f = functools.partial(
        pl.kernel,
        mesh=mesh,
        out_type=jax.ShapeDtypeStruct((nrows, _DIM), jnp.float32),
        scratch_types=[
            pltpu.VMEM((nchunk, _CHUNK), jnp.int32),
            pltpu.VMEM((gpad, _DIM), jnp.float32),
            pltpu.VMEM((bpw, _DIM), jnp.float32),
            pltpu.SemaphoreType.DMA,
        ],
        compiler_params=pltpu.CompilerParams(use_tc_tiling_on_sc=False),
    )(_make_emb_body(bpw, nchunk))
    return f(idx, table)


_RB = 64
_NRSTEP = _BATCH // _RB
_NBUF = 2


def _proj_body(emb_ref, w_ref, b_ref, out_ref):
    out_ref[...] = (
        lax.dot_general(
            emb_ref[...],
            w_ref[...],
            dimension_numbers=(((1,), (0,)), ((), ())),
            preferred_element_type=jnp.float32,
        )
        + b_ref[...]
    )


def _proj_body_alias(prev_ref, emb_ref, w_ref, b_ref, out_ref):
    _proj_body(emb_ref, w_ref, b_ref, out_ref)


def _project_rows(emb, wt, b2, prev, row0):
    nrows = emb.shape[0]
    nstep = nrows // _RB
    r0 = row0 // _RB
    cp = pltpu.CompilerParams(
        dimension_semantics=("arbitrary",),
        disable_bounds_checks=True,
    )
    common = dict(
        grid=(nstep,),
        out_shape=jax.ShapeDtypeStruct((_BATCH, _VOCAB), jnp.float32),
        out_specs=pl.BlockSpec((_RB, _VOCAB), lambda i: (r0 + i, 0)),
        compiler_params=cp,
    )
    emb_spec = pl.BlockSpec((_RB, _DIM), lambda i: (i, 0))
    w_spec = pl.BlockSpec((_DIM, _VOCAB), lambda i: (0, 0))
    b_spec = pl.BlockSpec((1, _VOCAB), lambda i: (0, 0))
    if prev is None:
        return pl.pallas_call(
            _proj_body,
            in_specs=[emb_spec, w_spec, b_spec],
            **common,
        )(emb, wt, b2)
    prev_spec = pl.BlockSpec(memory_space=pltpu.MemorySpace.HBM)
    return pl.pallas_call(
        _proj_body_alias,
        in_specs=[prev_spec, emb_spec, w_spec, b_spec],
        input_output_aliases={0: 0},
        **common,
    )(prev, emb, wt, b2)


def kernel(words, table, W, b):
    wt = W.T
    b2 = b.reshape(1, _VOCAB)
    emb = _embed(words, table)
    return _project_rows(emb, wt, b2, None, 0)
